# baseline (speedup 1.0000x reference)
import functools

import jax
import jax.numpy as jnp
from jax import lax
from jax.experimental import pallas as pl
from jax.experimental.pallas import tpu as pltpu
from jax.experimental.pallas import tpu_sc as plsc

_DIM = 770
_KQ = 65536
_B = 4096
_EMB = 768
_N = _DIM * _KQ
_NW = 32
_CH = _N // _NW


def _sc_copy_body(q_hbm, o_hbm):
    w = lax.axis_index("s") * 2 + lax.axis_index("c")
    base = pl.multiple_of(w * _CH, _CH)
    pltpu.sync_copy(q_hbm.at[pl.ds(base, _CH)], o_hbm.at[pl.ds(base, _CH)])


_sc_copy = functools.partial(
    pl.kernel,
    out_type=jax.ShapeDtypeStruct((_N,), jnp.float32),
    mesh=plsc.VectorSubcoreMesh(core_axis_name="c", subcore_axis_name="s"),
)(_sc_copy_body)


def _tc_update_body(pb_ref, emb_ref, extra_ref, q_ref, o_ref):
    del pb_ref, q_ref
    o_ref[0:_EMB, :] = emb_ref[...].T
    o_ref[_EMB:_DIM, :] = extra_ref[...]


def kernel(embedding_batch, CLabel, NumofLabel, queue, queue_ptr):
    n = embedding_batch.shape[0]
    idx = jnp.arange(n, dtype=jnp.float32) + (
        jnp.asarray(NumofLabel, dtype=jnp.float32) - jnp.float32(n)
    )
    extra = jnp.stack([CLabel.astype(jnp.float32), idx])

    ptr = queue_ptr[0]
    p = jnp.where(ptr + _B >= _KQ - 1, jnp.int32(0), ptr).astype(jnp.int32)
    pb = (p // _B).reshape(1)

    copied = _sc_copy(queue.reshape(_N)).reshape(_DIM, _KQ)

    grid_spec = pltpu.PrefetchScalarGridSpec(
        num_scalar_prefetch=1,
        grid=(1,),
        in_specs=[
            pl.BlockSpec((n, _EMB), lambda i, pb: (0, 0)),
            pl.BlockSpec((2, _B), lambda i, pb: (0, 0)),
            pl.BlockSpec(memory_space=pl.ANY),
        ],
        out_specs=pl.BlockSpec((_DIM, _B), lambda i, pb: (0, pb[0])),
    )

    out = pl.pallas_call(
        _tc_update_body,
        grid_spec=grid_spec,
        out_shape=jax.ShapeDtypeStruct((_DIM, _KQ), jnp.float32),
        input_output_aliases={3: 0},
    )(pb, embedding_batch, extra, copied)

    new_ptr = p + jnp.int32(_B)
    return (out, new_ptr)


# speedup vs baseline: 12.3769x; 12.3769x over previous
import functools

import jax
import jax.numpy as jnp
from jax import lax
from jax.experimental import pallas as pl
from jax.experimental.pallas import tpu as pltpu
from jax.experimental.pallas import tpu_sc as plsc

_DIM = 770
_KQ = 65536
_B = 4096
_EMB = 768
_N = _DIM * _KQ
_NW = 32
_CH = _N // _NW


_SUB = 45056
_ITERS = _CH // _SUB


def _sc_copy_body(q_hbm, o_hbm, buf, sem_in, sem_out):
    w = lax.axis_index("s") * 2 + lax.axis_index("c")
    base = pl.multiple_of(w * _CH, _CH)

    def src(k):
        return q_hbm.at[pl.ds(base + k * _SUB, _SUB)]

    def dst(k):
        return o_hbm.at[pl.ds(base + k * _SUB, _SUB)]

    pltpu.make_async_copy(src(0), buf.at[0], sem_in).start()

    def step(k, carry):
        b = lax.rem(k, 2)
        pltpu.make_async_copy(src(k), buf.at[b], sem_in).wait()
        pltpu.make_async_copy(buf.at[b], dst(k), sem_out).start()

        @pl.when(k >= 1)
        def _():
            pltpu.make_async_copy(buf.at[1 - b], dst(k - 1), sem_out).wait()

        @pl.when(k + 1 < _ITERS)
        def _():
            pltpu.make_async_copy(src(k + 1), buf.at[1 - b], sem_in).start()

        return carry

    lax.fori_loop(0, _ITERS, step, 0)
    last = _ITERS - 1
    pltpu.make_async_copy(buf.at[lax.rem(last, 2)], dst(last), sem_out).wait()


_sc_copy = functools.partial(
    pl.kernel,
    out_type=jax.ShapeDtypeStruct((_N,), jnp.float32),
    mesh=plsc.VectorSubcoreMesh(core_axis_name="c", subcore_axis_name="s"),
    scratch_types=[
        pltpu.VMEM((2, _SUB), jnp.float32),
        pltpu.SemaphoreType.DMA,
        pltpu.SemaphoreType.DMA,
    ],
)(_sc_copy_body)


def _tc_update_body(pb_ref, emb_ref, extra_ref, q_ref, o_ref):
    del pb_ref, q_ref
    o_ref[0:_EMB, :] = emb_ref[...].T
    o_ref[_EMB:_DIM, :] = extra_ref[...]


def kernel(embedding_batch, CLabel, NumofLabel, queue, queue_ptr):
    n = embedding_batch.shape[0]
    idx = jnp.arange(n, dtype=jnp.float32) + (
        jnp.asarray(NumofLabel, dtype=jnp.float32) - jnp.float32(n)
    )
    extra = jnp.stack([CLabel.astype(jnp.float32), idx])

    ptr = queue_ptr[0]
    p = jnp.where(ptr + _B >= _KQ - 1, jnp.int32(0), ptr).astype(jnp.int32)
    pb = (p // _B).reshape(1)

    copied = _sc_copy(queue.reshape(_N)).reshape(_DIM, _KQ)

    grid_spec = pltpu.PrefetchScalarGridSpec(
        num_scalar_prefetch=1,
        grid=(1,),
        in_specs=[
            pl.BlockSpec((n, _EMB), lambda i, pb: (0, 0)),
            pl.BlockSpec((2, _B), lambda i, pb: (0, 0)),
            pl.BlockSpec(memory_space=pl.ANY),
        ],
        out_specs=pl.BlockSpec((_DIM, _B), lambda i, pb: (0, pb[0])),
    )

    out = pl.pallas_call(
        _tc_update_body,
        grid_spec=grid_spec,
        out_shape=jax.ShapeDtypeStruct((_DIM, _KQ), jnp.float32),
        input_output_aliases={3: 0},
    )(pb, embedding_batch, extra, copied)

    new_ptr = p + jnp.int32(_B)
    return (out, new_ptr)


# speedup vs baseline: 45.7879x; 3.6995x over previous
import jax
import jax.numpy as jnp
from jax.experimental import pallas as pl
from jax.experimental.pallas import tpu as pltpu

_DIM = 770
_KQ = 65536
_B = 4096
_EMB = 768
_C = 1024
_NC = _KQ // _C
_UP = _B // _C


def _body(pb_ref, emb_ref, extra_ref, q_ref, o_ref):
    i = pl.program_id(0)
    c0 = pb_ref[0]
    is_upd = (i >= c0) & (i < c0 + _UP)

    @pl.when(is_upd)
    def _():
        o_ref[0:_EMB, :] = emb_ref[...].T
        off = pl.multiple_of((i - c0) * _C, _C)
        o_ref[_EMB:_DIM, :] = extra_ref[:, pl.ds(off, _C)]

    @pl.when(jnp.logical_not(is_upd))
    def _():
        o_ref[...] = q_ref[...]


def kernel(embedding_batch, CLabel, NumofLabel, queue, queue_ptr):
    n = embedding_batch.shape[0]
    idx = jnp.arange(n, dtype=jnp.float32) + (
        jnp.asarray(NumofLabel, dtype=jnp.float32) - jnp.float32(n)
    )
    extra = jnp.stack([CLabel.astype(jnp.float32), idx])

    ptr = queue_ptr[0]
    p = jnp.where(ptr + _B >= _KQ - 1, jnp.int32(0), ptr).astype(jnp.int32)
    pb = (p // _C).reshape(1)

    def emb_map(j, pb):
        return (jnp.clip(j - pb[0], 0, _UP - 1), 0)

    def q_map(j, pb):
        c0 = pb[0]
        is_upd = (j >= c0) & (j < c0 + _UP)
        return (0, jnp.where(is_upd, c0 + _UP, j))

    grid_spec = pltpu.PrefetchScalarGridSpec(
        num_scalar_prefetch=1,
        grid=(_NC,),
        in_specs=[
            pl.BlockSpec((_C, _EMB), emb_map),
            pl.BlockSpec((2, _B), lambda j, pb: (0, 0)),
            pl.BlockSpec((_DIM, _C), q_map),
        ],
        out_specs=pl.BlockSpec((_DIM, _C), lambda j, pb: (0, j)),
    )

    out = pl.pallas_call(
        _body,
        grid_spec=grid_spec,
        out_shape=jax.ShapeDtypeStruct((_DIM, _KQ), jnp.float32),
        compiler_params=pltpu.CompilerParams(
            dimension_semantics=("arbitrary",),
        ),
    )(pb, embedding_batch, extra, queue)

    new_ptr = p + jnp.int32(_B)
    return (out, new_ptr)


# speedup vs baseline: 47.9818x; 1.0479x over previous
import jax
import jax.numpy as jnp
from jax.experimental import pallas as pl
from jax.experimental.pallas import tpu as pltpu

_DIM = 770
_KQ = 65536
_B = 4096
_EMB = 768
_C = 2048
_NC = _KQ // _C
_UP = _B // _C


def _body(pb_ref, emb_ref, extra_ref, q_ref, o_ref):
    i = pl.program_id(0)
    c0 = pb_ref[0]
    is_upd = (i >= c0) & (i < c0 + _UP)

    @pl.when(is_upd)
    def _():
        o_ref[0:_EMB, :] = emb_ref[...].T
        off = pl.multiple_of((i - c0) * _C, _C)
        o_ref[_EMB:_DIM, :] = extra_ref[:, pl.ds(off, _C)]

    @pl.when(jnp.logical_not(is_upd))
    def _():
        o_ref[...] = q_ref[...]


def kernel(embedding_batch, CLabel, NumofLabel, queue, queue_ptr):
    n = embedding_batch.shape[0]
    idx = jnp.arange(n, dtype=jnp.float32) + (
        jnp.asarray(NumofLabel, dtype=jnp.float32) - jnp.float32(n)
    )
    extra = jnp.stack([CLabel.astype(jnp.float32), idx])

    ptr = queue_ptr[0]
    p = jnp.where(ptr + _B >= _KQ - 1, jnp.int32(0), ptr).astype(jnp.int32)
    pb = (p // _C).reshape(1)

    def emb_map(j, pb):
        return (jnp.clip(j - pb[0], 0, _UP - 1), 0)

    def q_map(j, pb):
        c0 = pb[0]
        is_upd = (j >= c0) & (j < c0 + _UP)
        return (0, jnp.where(is_upd, c0 + _UP, j))

    grid_spec = pltpu.PrefetchScalarGridSpec(
        num_scalar_prefetch=1,
        grid=(_NC,),
        in_specs=[
            pl.BlockSpec((_C, _EMB), emb_map),
            pl.BlockSpec((2, _B), lambda j, pb: (0, 0)),
            pl.BlockSpec((_DIM, _C), q_map),
        ],
        out_specs=pl.BlockSpec((_DIM, _C), lambda j, pb: (0, j)),
    )

    out = pl.pallas_call(
        _body,
        grid_spec=grid_spec,
        out_shape=jax.ShapeDtypeStruct((_DIM, _KQ), jnp.float32),
        compiler_params=pltpu.CompilerParams(
            dimension_semantics=("parallel",),
        ),
    )(pb, embedding_batch, extra, queue)

    new_ptr = p + jnp.int32(_B)
    return (out, new_ptr)
